# Initial kernel scaffold; baseline (speedup 1.0000x reference)
import jax
import jax.numpy as jnp
from jax.experimental import pallas as pl


def kernel(inputs, W0, b0, W1, b1, W2, b2, Wf, bf):
    raise NotImplementedError("write your pallas kernel here")


# speedup vs baseline: 7.6259x; 7.6259x over previous
import jax
import jax.numpy as jnp
from jax import lax
from jax.experimental import pallas as pl
from jax.experimental.pallas import tpu as pltpu

K = 16
TILE = 256


def _dot(a, b, dnums):
    return lax.dot_general(a, b, dnums, preferred_element_type=jnp.float32)


def _knn_edge_stage(f, W, b):
    C, N = f.shape
    ch = W.shape[0]
    Wa = W[:, :C]
    Wb = W[:, C:]
    cn = (((1,), (0,)), ((), ()))
    v = _dot(Wb, f, cn)
    u = _dot(Wa - Wb, f, cn) + b
    sq = jnp.sum(f * f, axis=0, keepdims=True)

    iota = lax.broadcasted_iota(jnp.int32, (TILE, N), 1)

    def extract(_, carry):
        d, macc = carry
        mn = jnp.min(d, axis=1, keepdims=True)
        tied = d <= mn
        jmin = jnp.min(jnp.where(tied, iota, N), axis=1, keepdims=True)
        onehot = (iota == jmin)
        vsel = _dot(v, onehot.astype(jnp.float32),
                    (((1,), (1,)), ((), ())))
        macc = jnp.maximum(macc, vsel)
        d = jnp.where(onehot, jnp.inf, d)
        return d, macc

    tiles = []
    for t in range(N // TILE):
        fr = f[:, t * TILE:(t + 1) * TILE]
        inner = _dot(fr, f, (((0,), (0,)), ((), ())))
        d = sq - 2.0 * inner
        _, macc = lax.fori_loop(
            0, K, extract,
            (d, jnp.full((ch, TILE), -jnp.inf, jnp.float32)))
        tiles.append(macc)
    m = jnp.concatenate(tiles, axis=1)
    return jax.nn.relu(u + m)


def _fwd(x_ref, W0_ref, b0_ref, W1_ref, b1_ref, W2_ref, b2_ref,
         Wf_ref, bf_ref, out_ref):
    x = x_ref[0]
    N = x.shape[1]
    f1 = _knn_edge_stage(x, W0_ref[...], b0_ref[...])
    f2 = _knn_edge_stage(f1, W1_ref[...], b1_ref[...]) + f1
    f3 = _knn_edge_stage(f2, W2_ref[...], b2_ref[...]) + f2

    Wf = Wf_ref[...]
    ch = f1.shape[0]
    cn = (((1,), (0,)), ((), ()))
    ff = (_dot(Wf[:, :ch], f1, cn) + _dot(Wf[:, ch:2 * ch], f2, cn)
          + _dot(Wf[:, 2 * ch:], f3, cn) + bf_ref[...])
    ff = jax.nn.relu(ff)
    fmax = jnp.max(ff, axis=1, keepdims=True)

    out_ref[0, 0:1024, :] = jnp.broadcast_to(fmax, (1024, N))
    out_ref[0, 1024:1088, :] = f1
    out_ref[0, 1088:1152, :] = f2
    out_ref[0, 1152:1216, :] = f3


def kernel(inputs, W0, b0, W1, b1, W2, b2, Wf, bf):
    x = inputs[..., 0]
    B, Cin, N = x.shape
    ch = W0.shape[0]
    b0c = b0.reshape(ch, 1)
    b1c = b1.reshape(ch, 1)
    b2c = b2.reshape(ch, 1)
    bfc = bf.reshape(-1, 1)
    Cout = Wf.shape[0] + 3 * ch

    full = lambda a: pl.BlockSpec(a.shape, lambda b: (0,) * a.ndim)
    out = pl.pallas_call(
        _fwd,
        grid=(B,),
        in_specs=[
            pl.BlockSpec((1, Cin, N), lambda b: (b, 0, 0)),
            full(W0), full(b0c), full(W1), full(b1c),
            full(W2), full(b2c), full(Wf), full(bfc),
        ],
        out_specs=pl.BlockSpec((1, Cout, N), lambda b: (b, 0, 0)),
        out_shape=jax.ShapeDtypeStruct((B, Cout, N), jnp.float32),
        compiler_params=pltpu.CompilerParams(
            dimension_semantics=("arbitrary",)),
    )(x, W0, b0c, W1, b1c, W2, b2c, Wf, bfc)
    return out[..., None]
